# baseline (speedup 1.0000x reference)
import jax
import jax.numpy as jnp
from jax.experimental import pallas as pl
from jax.experimental.pallas import tpu as pltpu
from jax.experimental.pallas import tpu_sc as plsc

_NC, _NS = 2, 16
_NW = _NC * _NS
_CH = 200
_NBUF = 4


def _sc_copy_body(e_hbm, eo_hbm, b0, b1, b2, b3, s0, s1, s2, s3, t0, t1, t2, t3):
    wid = jax.lax.axis_index("s") * _NC + jax.lax.axis_index("c")
    rows = e_hbm.shape[0]
    per_w = rows // _NW
    n = per_w // _CH
    base = wid * per_w
    bufs = (b0, b1, b2, b3)
    sins = (s0, s1, s2, s3)
    souts = (t0, t1, t2, t3)

    def ein(i):
        return pltpu.make_async_copy(
            e_hbm.at[pl.ds(base + i * _CH, _CH)], bufs[i % _NBUF], sins[i % _NBUF]
        )

    def eout(i):
        return pltpu.make_async_copy(
            bufs[i % _NBUF], eo_hbm.at[pl.ds(base + i * _CH, _CH)], souts[i % _NBUF]
        )

    ein(0).start()
    ein(1).start()
    for i in range(n):
        if i + 2 < n:
            if i >= 2:
                eout(i - 2).wait()
            ein(i + 2).start()
        ein(i).wait()
        eout(i).start()
    if n >= 2:
        eout(n - 2).wait()
    eout(n - 1).wait()


def _tc_copy_body(xb, xob):
    xob[...] = xb[...]


def kernel(x, edge_index, edge_attr):
    del edge_index

    sc_copy = pl.kernel(
        _sc_copy_body,
        out_type=jax.ShapeDtypeStruct(edge_attr.shape, edge_attr.dtype),
        mesh=plsc.VectorSubcoreMesh(core_axis_name="c", subcore_axis_name="s"),
        scratch_types=(
            [pltpu.VMEM((_CH, edge_attr.shape[1]), edge_attr.dtype)] * _NBUF
            + [pltpu.SemaphoreType.DMA] * (2 * _NBUF)
        ),
    )
    eo = sc_copy(edge_attr)

    grid = 10
    xb = x.shape[0] // grid
    xo = pl.pallas_call(
        _tc_copy_body,
        grid=(grid,),
        in_specs=[pl.BlockSpec((xb, x.shape[1]), lambda i: (i, 0))],
        out_specs=pl.BlockSpec((xb, x.shape[1]), lambda i: (i, 0)),
        out_shape=jax.ShapeDtypeStruct(x.shape, x.dtype),
    )(x)
    return (xo, eo)
